# Initial kernel scaffold; baseline (speedup 1.0000x reference)
import functools

import jax
import jax.numpy as jnp
from jax import lax
from jax.experimental import pallas as pl
from jax.experimental.pallas import tpu as pltpu
from jax.experimental.pallas import tpu_sc as plsc

N = 160000
D = 256
NB = 96
DUMP = 93
CH = 128
CHUNKS = N // CH
NS = 16
NC = 2
COLS = D // NC
RPT = NB // NS
L = 16


def _sc_segment_mean(features, labels):
    mesh = plsc.VectorSubcoreMesh(core_axis_name="c", subcore_axis_name="s")

    @functools.partial(
        pl.kernel,
        out_type=jax.ShapeDtypeStruct((NB, D), jnp.float32),
        mesh=mesh,
        scratch_types=[
            pltpu.VMEM((CH, COLS), jnp.float32),
            pltpu.VMEM((CH,), jnp.int32),
            pltpu.VMEM((CH,), jnp.int32),
            pltpu.VMEM((CH, L), jnp.float32),
            pltpu.VMEM((RPT, COLS), jnp.float32),
            pltpu.VMEM((RPT, L), jnp.float32),
            pltpu.VMEM_SHARED((NB, COLS), jnp.float32),
            pltpu.VMEM_SHARED((NB, L), jnp.float32),
        ],
    )
    def body(feat_hbm, lab_hbm, out_hbm, feat_buf, lab_buf, ids_buf,
             ones_buf, tmp, tmpc, accum, counts):
        cid = lax.axis_index("c")
        sid = lax.axis_index("s")
        col0 = pl.multiple_of(cid * COLS, COLS)
        r0 = sid * RPT

        zero16 = jnp.zeros((L,), jnp.float32)
        one16 = jnp.ones((L,), jnp.float32)
        for i in range(RPT):
            for k in range(COLS // L):
                tmp[i, pl.ds(k * L, L)] = zero16
            tmpc[i, pl.ds(0, L)] = zero16
        pltpu.sync_copy(tmp, accum.at[pl.ds(r0, RPT), :])
        pltpu.sync_copy(tmpc, counts.at[pl.ds(r0, RPT), :])
        for i in range(CH):
            ones_buf[i, pl.ds(0, L)] = one16
        plsc.subcore_barrier()

        nk = (CHUNKS - 1 - sid) // NS + 1

        def step(k, carry):
            c = sid + k * NS
            base = pl.multiple_of(c * CH, CH)
            pltpu.sync_copy(feat_hbm.at[pl.ds(base, CH), pl.ds(col0, COLS)],
                            feat_buf)
            pltpu.sync_copy(lab_hbm.at[pl.ds(base, CH)], lab_buf)
            for j in range(CH // L):
                v = lab_buf[pl.ds(j * L, L)]
                ids = v - 3
                ids = jnp.where((ids < 0) | (ids > DUMP - 1), DUMP, ids)
                ids_buf[pl.ds(j * L, L)] = ids
            pltpu.sync_copy(feat_buf, accum.at[ids_buf], add=True)
            pltpu.sync_copy(ones_buf, counts.at[ids_buf], add=True)
            return carry

        lax.fori_loop(0, nk, step, 0)
        plsc.subcore_barrier()

        pltpu.sync_copy(accum.at[pl.ds(r0, RPT), :], tmp)
        pltpu.sync_copy(counts.at[pl.ds(r0, RPT), :], tmpc)
        for i in range(RPT):
            cnt = tmpc[i, pl.ds(0, L)]
            pos = cnt > 0.0
            den = jnp.where(pos, cnt, 1.0)
            for k in range(COLS // L):
                v = tmp[i, pl.ds(k * L, L)]
                tmp[i, pl.ds(k * L, L)] = jnp.where(pos, v / den, 0.0)
        pltpu.sync_copy(tmp, out_hbm.at[pl.ds(r0, RPT), pl.ds(col0, COLS)])

    return body(features, labels)


def kernel(features, labels):
    center = _sc_segment_mean(features, labels)
    return center[: DUMP]


# speedup vs baseline: 4.0208x; 4.0208x over previous
import functools

import jax
import jax.numpy as jnp
from jax import lax
from jax.experimental import pallas as pl
from jax.experimental.pallas import tpu as pltpu
from jax.experimental.pallas import tpu_sc as plsc

N = 160000
D = 256
NB = 96
DUMP = 93
CH = 128
CHUNKS = N // CH
NS = 16
NC = 2
COLS = D // NC
RPT = 8
NFT = NB // RPT
L = 16


def _sc_segment_mean(features, labels):
    mesh = plsc.VectorSubcoreMesh(core_axis_name="c", subcore_axis_name="s")

    @functools.partial(
        pl.kernel,
        out_type=jax.ShapeDtypeStruct((NB, D), jnp.float32),
        mesh=mesh,
        scratch_types=[
            pltpu.VMEM((CH, COLS), jnp.float32),
            pltpu.VMEM((CH,), jnp.int32),
            pltpu.VMEM((CH,), jnp.int32),
            pltpu.VMEM((CH, COLS), jnp.float32),
            pltpu.VMEM((RPT, COLS), jnp.float32),
            pltpu.VMEM((RPT, COLS), jnp.float32),
            pltpu.VMEM_SHARED((NB, COLS), jnp.float32),
            pltpu.VMEM_SHARED((NB, COLS), jnp.float32),
        ],
    )
    def body(feat_hbm, lab_hbm, out_hbm, feat_buf, lab_buf, ids_buf,
             ones_buf, tmp, tmpc, accum, counts):
        cid = lax.axis_index("c")
        sid = lax.axis_index("s")
        col0 = pl.multiple_of(cid * COLS, COLS)
        r0 = pl.multiple_of(sid * RPT, RPT)

        zero16 = jnp.zeros((L,), jnp.float32)
        one16 = jnp.ones((L,), jnp.float32)
        for i in range(RPT):
            for k in range(COLS // L):
                tmp[i, pl.ds(k * L, L)] = zero16
                tmpc[i, pl.ds(k * L, L)] = zero16

        @pl.when(sid < NFT)
        def _init():
            pltpu.sync_copy(tmp, accum.at[pl.ds(r0, RPT), :])
            pltpu.sync_copy(tmpc, counts.at[pl.ds(r0, RPT), :])

        for i in range(CH):
            for k in range(COLS // L):
                ones_buf[i, pl.ds(k * L, L)] = one16
        plsc.subcore_barrier()

        nk = (CHUNKS - 1 - sid) // NS + 1

        def step(k, carry):
            c = sid + k * NS
            base = pl.multiple_of(c * CH, CH)
            pltpu.sync_copy(feat_hbm.at[pl.ds(base, CH), pl.ds(col0, COLS)],
                            feat_buf)
            pltpu.sync_copy(lab_hbm.at[pl.ds(base, CH)], lab_buf)
            for j in range(CH // L):
                v = lab_buf[pl.ds(j * L, L)]
                ids = v - 3
                ids = jnp.where((ids < 0) | (ids > DUMP - 1), DUMP, ids)
                ids_buf[pl.ds(j * L, L)] = ids
            pltpu.sync_copy(feat_buf, accum.at[ids_buf], add=True)
            pltpu.sync_copy(ones_buf, counts.at[ids_buf], add=True)
            return carry

        lax.fori_loop(0, nk, step, 0)
        plsc.subcore_barrier()

        @pl.when(sid < NFT)
        def _finalize():
            pltpu.sync_copy(accum.at[pl.ds(r0, RPT), :], tmp)
            pltpu.sync_copy(counts.at[pl.ds(r0, RPT), :], tmpc)
            for i in range(RPT):
                cnt = tmpc[i, pl.ds(0, L)]
                pos = cnt > 0.0
                den = jnp.where(pos, cnt, 1.0)
                for k in range(COLS // L):
                    v = tmp[i, pl.ds(k * L, L)]
                    tmp[i, pl.ds(k * L, L)] = jnp.where(pos, v / den, 0.0)
            pltpu.sync_copy(tmp, out_hbm.at[pl.ds(r0, RPT), pl.ds(col0, COLS)])

    return body(features, labels)


def kernel(features, labels):
    center = _sc_segment_mean(features, labels)
    return center[: DUMP]


# speedup vs baseline: 6.5253x; 1.6229x over previous
import functools

import jax
import jax.numpy as jnp
from jax import lax
from jax.experimental import pallas as pl
from jax.experimental.pallas import tpu as pltpu
from jax.experimental.pallas import tpu_sc as plsc

N = 160000
D = 256
NB = 96
DUMP = 93
CH = 128
NS = 16
NC = 2
COLS = D // NC
RPT = 8
NFT = NB // RPT
L = 16
NK = (N // CH) // NS
TROWS = NK * CH
TAIL = N - NS * TROWS


def _sc_segment_mean(features, labels):
    mesh = plsc.VectorSubcoreMesh(core_axis_name="c", subcore_axis_name="s")

    @functools.partial(
        pl.kernel,
        out_type=jax.ShapeDtypeStruct((NB, D), jnp.float32),
        mesh=mesh,
        scratch_types=[
            pltpu.VMEM((CH, COLS), jnp.float32),
            pltpu.VMEM((CH, COLS), jnp.float32),
            pltpu.VMEM((CH,), jnp.int32),
            pltpu.VMEM((CH,), jnp.int32),
            pltpu.VMEM((TROWS,), jnp.int32),
            pltpu.VMEM((CH, COLS), jnp.float32),
            pltpu.VMEM((RPT, COLS), jnp.float32),
            pltpu.VMEM((RPT, COLS), jnp.float32),
            pltpu.VMEM_SHARED((NB, COLS), jnp.float32),
            pltpu.VMEM_SHARED((NB, COLS), jnp.float32),
            pltpu.SemaphoreType.DMA,
            pltpu.SemaphoreType.DMA,
            pltpu.SemaphoreType.DMA,
            pltpu.SemaphoreType.DMA,
            pltpu.SemaphoreType.DMA,
            pltpu.SemaphoreType.DMA,
            pltpu.SemaphoreType.DMA,
        ],
    )
    def body(feat_hbm, lab_hbm, out_hbm, f0, f1, ids0, ids1, lab_all,
             ones_buf, tmp, tmpc, accum, counts,
             sf0, sf1, ss0, ss1, so0, so1, sl):
        cid = lax.axis_index("c")
        sid = lax.axis_index("s")
        col0 = pl.multiple_of(cid * COLS, COLS)
        r0 = pl.multiple_of(sid * RPT, RPT)
        tbase = pl.multiple_of(sid * TROWS, CH)

        zero16 = jnp.zeros((L,), jnp.float32)
        one16 = jnp.ones((L,), jnp.float32)
        for i in range(RPT):
            for k in range(COLS // L):
                tmp[i, pl.ds(k * L, L)] = zero16
                tmpc[i, pl.ds(k * L, L)] = zero16

        @pl.when(sid < NFT)
        def _init():
            pltpu.sync_copy(tmp, accum.at[pl.ds(r0, RPT), :])
            pltpu.sync_copy(tmpc, counts.at[pl.ds(r0, RPT), :])

        pltpu.async_copy(lab_hbm.at[pl.ds(tbase, TROWS)], lab_all, sl)

        def fill_ones(i, carry):
            for k in range(COLS // L):
                ones_buf[i, pl.ds(k * L, L)] = one16
            return carry
        lax.fori_loop(0, CH, fill_ones, 0)

        plsc.subcore_barrier()
        pltpu.make_async_copy(lab_hbm.at[pl.ds(tbase, TROWS)], lab_all,
                              sl).wait()

        def feat_src(k):
            base = pl.multiple_of((sid * NK + k) * CH, CH)
            return feat_hbm.at[pl.ds(base, CH), pl.ds(col0, COLS)]

        def issue_in(k, fb, sf):
            pltpu.async_copy(feat_src(k), fb, sf)

        def wait_in(k, fb, sf):
            pltpu.make_async_copy(feat_src(k), fb, sf).wait()

        def ids_of(k, ib):
            for j in range(CH // L):
                v = lab_all[pl.ds(k * CH + j * L, L)]
                ids = v - 3
                ids = jnp.where((ids < 0) | (ids > DUMP - 1), DUMP, ids)
                ib[pl.ds(j * L, L)] = ids

        def issue_sc(fb, ib, ss, so):
            pltpu.async_copy(fb, accum.at[ib], ss, add=True)
            pltpu.async_copy(ones_buf, counts.at[ib], so, add=True)

        def wait_sc(fb, ib, ss, so):
            pltpu.make_async_copy(fb, accum.at[ib], ss).wait()
            pltpu.make_async_copy(ones_buf, counts.at[ib], so).wait()

        issue_in(0, f0, sf0)
        issue_in(1, f1, sf1)
        wait_in(0, f0, sf0)
        ids_of(0, ids0)
        issue_sc(f0, ids0, ss0, so0)

        def pairbody(kk, carry):
            k1 = 2 * kk + 1
            k2 = 2 * kk + 2
            wait_in(k1, f1, sf1)
            ids_of(k1, ids1)
            issue_sc(f1, ids1, ss1, so1)
            wait_sc(f0, ids0, ss0, so0)
            issue_in(k2, f0, sf0)
            wait_in(k2, f0, sf0)
            ids_of(k2, ids0)
            issue_sc(f0, ids0, ss0, so0)
            wait_sc(f1, ids1, ss1, so1)
            issue_in(k2 + 1, f1, sf1)
            return carry

        lax.fori_loop(0, (NK - 2) // 2, pairbody, 0)

        wait_in(NK - 1, f1, sf1)
        ids_of(NK - 1, ids1)
        issue_sc(f1, ids1, ss1, so1)
        wait_sc(f0, ids0, ss0, so0)
        wait_sc(f1, ids1, ss1, so1)

        @pl.when(sid < TAIL // CH)
        def _tail():
            base = pl.multiple_of(NS * TROWS + sid * CH, CH)
            pltpu.sync_copy(feat_hbm.at[pl.ds(base, CH), pl.ds(col0, COLS)],
                            f0)
            pltpu.sync_copy(lab_hbm.at[pl.ds(base, CH)],
                            lab_all.at[pl.ds(0, CH)])
            ids_of(0, ids0)
            pltpu.sync_copy(f0, accum.at[ids0], add=True)
            pltpu.sync_copy(ones_buf, counts.at[ids0], add=True)

        plsc.subcore_barrier()

        @pl.when(sid < NFT)
        def _finalize():
            pltpu.sync_copy(accum.at[pl.ds(r0, RPT), :], tmp)
            pltpu.sync_copy(counts.at[pl.ds(r0, RPT), :], tmpc)
            for i in range(RPT):
                cnt = tmpc[i, pl.ds(0, L)]
                pos = cnt > 0.0
                den = jnp.where(pos, cnt, 1.0)
                for k in range(COLS // L):
                    v = tmp[i, pl.ds(k * L, L)]
                    tmp[i, pl.ds(k * L, L)] = jnp.where(pos, v / den, 0.0)
            pltpu.sync_copy(tmp, out_hbm.at[pl.ds(r0, RPT), pl.ds(col0, COLS)])

    return body(features, labels)


def kernel(features, labels):
    center = _sc_segment_mean(features, labels)
    return center[: DUMP]


# speedup vs baseline: 8.0184x; 1.2288x over previous
import functools

import jax
import jax.numpy as jnp
from jax import lax
from jax.experimental import pallas as pl
from jax.experimental.pallas import tpu as pltpu
from jax.experimental.pallas import tpu_sc as plsc

N = 160000
D = 256
NB = 96
DUMP = 93
CH = 128
NS = 16
NC = 2
COLS = D // NC
RPT = 8
NFT = NB // RPT
L = 16
NK = (N // CH) // NS
TROWS = NK * CH
TAIL = N - NS * TROWS
BPG = 8


def _sc_segment_sum(features, labels):
    mesh = plsc.VectorSubcoreMesh(core_axis_name="c", subcore_axis_name="s")

    @functools.partial(
        pl.kernel,
        out_type=jax.ShapeDtypeStruct((NB, D), jnp.float32),
        mesh=mesh,
        scratch_types=[
            pltpu.VMEM((CH, COLS), jnp.float32),
            pltpu.VMEM((CH, COLS), jnp.float32),
            pltpu.VMEM((CH,), jnp.int32),
            pltpu.VMEM((CH,), jnp.int32),
            pltpu.VMEM((TROWS,), jnp.int32),
            pltpu.VMEM((RPT, COLS), jnp.float32),
            pltpu.VMEM_SHARED((NB, COLS), jnp.float32),
            pltpu.SemaphoreType.DMA,
            pltpu.SemaphoreType.DMA,
            pltpu.SemaphoreType.DMA,
            pltpu.SemaphoreType.DMA,
            pltpu.SemaphoreType.DMA,
        ],
    )
    def body(feat_hbm, lab_hbm, out_hbm, f0, f1, ids0, ids1, lab_all,
             tmp, accum, sf0, sf1, ss0, ss1, sl):
        cid = lax.axis_index("c")
        sid = lax.axis_index("s")
        col0 = pl.multiple_of(cid * COLS, COLS)
        r0 = pl.multiple_of(sid * RPT, RPT)
        tbase = pl.multiple_of(sid * TROWS, CH)

        zero16 = jnp.zeros((L,), jnp.float32)
        for i in range(RPT):
            for k in range(COLS // L):
                tmp[i, pl.ds(k * L, L)] = zero16

        @pl.when(sid < NFT)
        def _init():
            pltpu.sync_copy(tmp, accum.at[pl.ds(r0, RPT), :])

        pltpu.async_copy(lab_hbm.at[pl.ds(tbase, TROWS)], lab_all, sl)

        plsc.subcore_barrier()
        pltpu.make_async_copy(lab_hbm.at[pl.ds(tbase, TROWS)], lab_all,
                              sl).wait()

        def feat_src(k):
            base = pl.multiple_of((sid * NK + k) * CH, CH)
            return feat_hbm.at[pl.ds(base, CH), pl.ds(col0, COLS)]

        def issue_in(k, fb, sf):
            pltpu.async_copy(feat_src(k), fb, sf)

        def wait_in(k, fb, sf):
            pltpu.make_async_copy(feat_src(k), fb, sf).wait()

        def ids_of(k, ib):
            for j in range(CH // L):
                v = lab_all[pl.ds(k * CH + j * L, L)]
                ids = v - 3
                ids = jnp.where((ids < 0) | (ids > DUMP - 1), DUMP, ids)
                ib[pl.ds(j * L, L)] = ids

        def issue_sc(fb, ib, ss):
            pltpu.async_copy(fb, accum.at[ib], ss, add=True)

        def wait_sc(fb, ib, ss):
            pltpu.make_async_copy(fb, accum.at[ib], ss).wait()

        issue_in(0, f0, sf0)
        issue_in(1, f1, sf1)
        wait_in(0, f0, sf0)
        ids_of(0, ids0)
        issue_sc(f0, ids0, ss0)

        def pairbody(kk, carry):
            k1 = 2 * kk + 1
            k2 = 2 * kk + 2
            wait_in(k1, f1, sf1)
            ids_of(k1, ids1)
            issue_sc(f1, ids1, ss1)
            wait_sc(f0, ids0, ss0)
            issue_in(k2, f0, sf0)
            wait_in(k2, f0, sf0)
            ids_of(k2, ids0)
            issue_sc(f0, ids0, ss0)
            wait_sc(f1, ids1, ss1)
            issue_in(k2 + 1, f1, sf1)
            return carry

        lax.fori_loop(0, (NK - 2) // 2, pairbody, 0)

        wait_in(NK - 1, f1, sf1)
        ids_of(NK - 1, ids1)
        issue_sc(f1, ids1, ss1)
        wait_sc(f0, ids0, ss0)
        wait_sc(f1, ids1, ss1)

        @pl.when(sid < TAIL // CH)
        def _tail():
            base = pl.multiple_of(NS * TROWS + sid * CH, CH)
            pltpu.sync_copy(feat_hbm.at[pl.ds(base, CH), pl.ds(col0, COLS)],
                            f0)
            pltpu.sync_copy(lab_hbm.at[pl.ds(base, CH)],
                            lab_all.at[pl.ds(0, CH)])
            ids_of(0, ids0)
            pltpu.sync_copy(f0, accum.at[ids0], add=True)

        plsc.subcore_barrier()

        @pl.when(sid < NFT)
        def _finalize():
            pltpu.sync_copy(accum.at[pl.ds(r0, RPT), :],
                            out_hbm.at[pl.ds(r0, RPT), pl.ds(col0, COLS)])

    return body(features, labels)


def _histo_kernel(lab_ref, out_ref):
    g = pl.program_id(0)
    lab = lab_ref[...]
    rows = []
    for i in range(BPG):
        b = g * BPG + i
        cnt = jnp.sum((lab == (b + 3)).astype(jnp.float32))
        rows.append(jnp.full((D,), cnt, jnp.float32))
    out_ref[...] = jnp.stack(rows, axis=0)


def _tc_histogram(labels2d):
    return pl.pallas_call(
        _histo_kernel,
        grid=(NB // BPG,),
        in_specs=[pl.BlockSpec(labels2d.shape, lambda g: (0, 0))],
        out_specs=pl.BlockSpec((BPG, D), lambda g: (g, 0)),
        out_shape=jax.ShapeDtypeStruct((NB, D), jnp.float32),
    )(labels2d)


def _div_kernel(sum_ref, cnt_ref, out_ref):
    s = sum_ref[...]
    c = cnt_ref[...]
    out_ref[...] = jnp.where(c > 0.0, s / jnp.where(c > 0.0, c, 1.0), 0.0)


def _tc_divide(sums, cnts):
    return pl.pallas_call(
        _div_kernel,
        in_specs=[pl.BlockSpec(sums.shape, lambda: (0, 0)),
                  pl.BlockSpec(cnts.shape, lambda: (0, 0))],
        out_specs=pl.BlockSpec(sums.shape, lambda: (0, 0)),
        out_shape=jax.ShapeDtypeStruct(sums.shape, jnp.float32),
    )(sums, cnts)


def kernel(features, labels):
    sums = _sc_segment_sum(features, labels)
    cnts = _tc_histogram(labels.reshape(N // CH, CH))
    center = _tc_divide(sums, cnts)
    return center[:DUMP]


# speedup vs baseline: 8.0266x; 1.0010x over previous
import functools

import jax
import jax.numpy as jnp
from jax import lax
from jax.experimental import pallas as pl
from jax.experimental.pallas import tpu as pltpu
from jax.experimental.pallas import tpu_sc as plsc

N = 160000
D = 256
NB = 96
DUMP = 93
CH = 128
NS = 16
NC = 2
COLS = D // NC
RPT = 8
NFT = NB // RPT
L = 16
NK = (N // CH) // NS
TROWS = NK * CH
TAIL = N - NS * TROWS
BPG = 8


def _sc_segment_sum(features, labels):
    mesh = plsc.VectorSubcoreMesh(core_axis_name="c", subcore_axis_name="s")

    @functools.partial(
        pl.kernel,
        out_type=jax.ShapeDtypeStruct((NB, D), jnp.float32),
        mesh=mesh,
        scratch_types=[
            pltpu.VMEM((CH, COLS), jnp.float32),
            pltpu.VMEM((CH, COLS), jnp.float32),
            pltpu.VMEM((TROWS,), jnp.int32),
            pltpu.VMEM((RPT, COLS), jnp.float32),
            pltpu.VMEM_SHARED((NB, COLS), jnp.float32),
            pltpu.SemaphoreType.DMA,
            pltpu.SemaphoreType.DMA,
            pltpu.SemaphoreType.DMA,
            pltpu.SemaphoreType.DMA,
            pltpu.SemaphoreType.DMA,
        ],
    )
    def body(feat_hbm, lab_hbm, out_hbm, f0, f1, lab_all,
             tmp, accum, sf0, sf1, ss0, ss1, sl):
        cid = lax.axis_index("c")
        sid = lax.axis_index("s")
        col0 = pl.multiple_of(cid * COLS, COLS)
        r0 = pl.multiple_of(sid * RPT, RPT)
        tbase = pl.multiple_of(sid * TROWS, CH)

        zero16 = jnp.zeros((L,), jnp.float32)
        for i in range(RPT):
            for k in range(COLS // L):
                tmp[i, pl.ds(k * L, L)] = zero16

        @pl.when(sid < NFT)
        def _init():
            pltpu.sync_copy(tmp, accum.at[pl.ds(r0, RPT), :])

        pltpu.async_copy(lab_hbm.at[pl.ds(tbase, TROWS)], lab_all, sl)

        plsc.subcore_barrier()
        pltpu.make_async_copy(lab_hbm.at[pl.ds(tbase, TROWS)], lab_all,
                              sl).wait()

        def remap(i, carry):
            v = lab_all[pl.ds(i * L, L)]
            ids = v - 3
            ids = jnp.where((ids < 0) | (ids > DUMP - 1), DUMP, ids)
            lab_all[pl.ds(i * L, L)] = ids
            return carry

        lax.fori_loop(0, TROWS // L, remap, 0)

        def feat_src(k):
            base = pl.multiple_of((sid * NK + k) * CH, CH)
            return feat_hbm.at[pl.ds(base, CH), pl.ds(col0, COLS)]

        def issue_in(k, fb, sf):
            pltpu.async_copy(feat_src(k), fb, sf)

        def wait_in(k, fb, sf):
            pltpu.make_async_copy(feat_src(k), fb, sf).wait()

        def ids_at(k):
            return lab_all.at[pl.ds(k * CH, CH)]

        def issue_sc(fb, k, ss):
            pltpu.async_copy(fb, accum.at[ids_at(k)], ss, add=True)

        def wait_sc(fb, k, ss):
            pltpu.make_async_copy(fb, accum.at[ids_at(k)], ss).wait()

        issue_in(0, f0, sf0)
        issue_in(1, f1, sf1)
        wait_in(0, f0, sf0)
        issue_sc(f0, 0, ss0)

        def pairbody(kk, carry):
            k1 = 2 * kk + 1
            k2 = 2 * kk + 2
            wait_in(k1, f1, sf1)
            issue_sc(f1, k1, ss1)
            wait_sc(f0, k1 - 1, ss0)
            issue_in(k2, f0, sf0)
            wait_in(k2, f0, sf0)
            issue_sc(f0, k2, ss0)
            wait_sc(f1, k1, ss1)
            issue_in(k2 + 1, f1, sf1)
            return carry

        lax.fori_loop(0, (NK - 2) // 2, pairbody, 0)

        wait_in(NK - 1, f1, sf1)
        issue_sc(f1, NK - 1, ss1)
        wait_sc(f0, NK - 2, ss0)
        wait_sc(f1, NK - 1, ss1)

        @pl.when(sid < TAIL // CH)
        def _tail():
            base = pl.multiple_of(NS * TROWS + sid * CH, CH)
            pltpu.sync_copy(feat_hbm.at[pl.ds(base, CH), pl.ds(col0, COLS)],
                            f0)
            pltpu.sync_copy(lab_hbm.at[pl.ds(base, CH)],
                            lab_all.at[pl.ds(0, CH)])
            lax.fori_loop(0, CH // L, remap, 0)
            pltpu.sync_copy(f0, accum.at[ids_at(0)], add=True)

        plsc.subcore_barrier()

        @pl.when(sid < NFT)
        def _finalize():
            pltpu.sync_copy(accum.at[pl.ds(r0, RPT), :],
                            out_hbm.at[pl.ds(r0, RPT), pl.ds(col0, COLS)])

    return body(features, labels)


def _histo_kernel(lab_ref, out_ref):
    g = pl.program_id(0)
    lab = lab_ref[...]
    rows = []
    for i in range(BPG):
        b = g * BPG + i
        cnt = jnp.sum((lab == (b + 3)).astype(jnp.float32))
        rows.append(jnp.full((D,), cnt, jnp.float32))
    out_ref[...] = jnp.stack(rows, axis=0)


def _tc_histogram(labels2d):
    return pl.pallas_call(
        _histo_kernel,
        grid=(NB // BPG,),
        in_specs=[pl.BlockSpec(labels2d.shape, lambda g: (0, 0))],
        out_specs=pl.BlockSpec((BPG, D), lambda g: (g, 0)),
        out_shape=jax.ShapeDtypeStruct((NB, D), jnp.float32),
    )(labels2d)


def _div_kernel(sum_ref, cnt_ref, out_ref):
    s = sum_ref[...]
    c = cnt_ref[...]
    out_ref[...] = jnp.where(c > 0.0, s / jnp.where(c > 0.0, c, 1.0), 0.0)


def _tc_divide(sums, cnts):
    return pl.pallas_call(
        _div_kernel,
        in_specs=[pl.BlockSpec(sums.shape, lambda: (0, 0)),
                  pl.BlockSpec(cnts.shape, lambda: (0, 0))],
        out_specs=pl.BlockSpec(sums.shape, lambda: (0, 0)),
        out_shape=jax.ShapeDtypeStruct(sums.shape, jnp.float32),
    )(sums, cnts)


def kernel(features, labels):
    sums = _sc_segment_sum(features, labels)
    cnts = _tc_histogram(labels.reshape(N // CH, CH))
    center = _tc_divide(sums, cnts)
    return center[:DUMP]


# speedup vs baseline: 9.1274x; 1.1371x over previous
import functools

import jax
import jax.numpy as jnp
from jax import lax
from jax.experimental import pallas as pl
from jax.experimental.pallas import tpu as pltpu
from jax.experimental.pallas import tpu_sc as plsc

N = 160000
D = 256
NB = 96
DUMP = 93
CH = 128
NS = 16
NC = 2
COLS = D // NC
RPT = 8
NFT = NB // RPT
L = 16
NK = (N // CH) // NS
TROWS = NK * CH
TAIL = N - NS * TROWS
BPG = 8


def _sc_segment_sum(features, labels):
    mesh = plsc.VectorSubcoreMesh(core_axis_name="c", subcore_axis_name="s")

    @functools.partial(
        pl.kernel,
        out_type=jax.ShapeDtypeStruct((NB, D), jnp.float32),
        mesh=mesh,
        scratch_types=[
            pltpu.VMEM((CH, COLS), jnp.float32),
            pltpu.VMEM((CH, COLS), jnp.float32),
            pltpu.VMEM((CH, COLS), jnp.float32),
            pltpu.VMEM((CH, COLS), jnp.float32),
            pltpu.VMEM((TROWS,), jnp.int32),
            pltpu.VMEM((RPT, COLS), jnp.float32),
            pltpu.VMEM_SHARED((NB, COLS), jnp.float32),
            pltpu.SemaphoreType.DMA,
            pltpu.SemaphoreType.DMA,
            pltpu.SemaphoreType.DMA,
            pltpu.SemaphoreType.DMA,
            pltpu.SemaphoreType.DMA,
            pltpu.SemaphoreType.DMA,
            pltpu.SemaphoreType.DMA,
            pltpu.SemaphoreType.DMA,
            pltpu.SemaphoreType.DMA,
        ],
    )
    def body(feat_hbm, lab_hbm, out_hbm, f0, f1, f2, f3, lab_all,
             tmp, accum, sf0, sf1, sf2, sf3, ss0, ss1, ss2, ss3, sl):
        cid = lax.axis_index("c")
        sid = lax.axis_index("s")
        col0 = pl.multiple_of(cid * COLS, COLS)
        r0 = pl.multiple_of(sid * RPT, RPT)
        tbase = pl.multiple_of(sid * TROWS, CH)

        zero16 = jnp.zeros((L,), jnp.float32)
        for i in range(RPT):
            for k in range(COLS // L):
                tmp[i, pl.ds(k * L, L)] = zero16

        @pl.when(sid < NFT)
        def _init():
            pltpu.sync_copy(tmp, accum.at[pl.ds(r0, RPT), :])

        pltpu.async_copy(lab_hbm.at[pl.ds(tbase, TROWS)], lab_all, sl)

        plsc.subcore_barrier()
        pltpu.make_async_copy(lab_hbm.at[pl.ds(tbase, TROWS)], lab_all,
                              sl).wait()

        def remap(i, carry):
            v = lab_all[pl.ds(i * L, L)]
            ids = v - 3
            ids = jnp.where((ids < 0) | (ids > DUMP - 1), DUMP, ids)
            lab_all[pl.ds(i * L, L)] = ids
            return carry

        lax.fori_loop(0, TROWS // L, remap, 0)

        def feat_src(k):
            base = pl.multiple_of((sid * NK + k) * CH, CH)
            return feat_hbm.at[pl.ds(base, CH), pl.ds(col0, COLS)]

        def issue_in(k, fb, sf):
            pltpu.async_copy(feat_src(k), fb, sf)

        def wait_in(k, fb, sf):
            pltpu.make_async_copy(feat_src(k), fb, sf).wait()

        def ids_at(k):
            return lab_all.at[pl.ds(k * CH, CH)]

        def issue_sc(fb, k, ss):
            pltpu.async_copy(fb, accum.at[ids_at(k)], ss, add=True)

        def wait_sc(fb, k, ss):
            pltpu.make_async_copy(fb, accum.at[ids_at(k)], ss).wait()

        issue_in(0, f0, sf0)
        issue_in(1, f1, sf1)
        issue_in(2, f2, sf2)
        issue_in(3, f3, sf3)

        def quadbody(kk, carry):
            k = 4 * kk
            wait_in(k, f0, sf0)
            issue_sc(f0, k, ss0)
            wait_in(k + 1, f1, sf1)
            issue_sc(f1, k + 1, ss1)
            wait_in(k + 2, f2, sf2)
            issue_sc(f2, k + 2, ss2)
            wait_in(k + 3, f3, sf3)
            issue_sc(f3, k + 3, ss3)
            wait_sc(f0, k, ss0)
            issue_in(k + 4, f0, sf0)
            wait_sc(f1, k + 1, ss1)
            issue_in(k + 5, f1, sf1)
            wait_sc(f2, k + 2, ss2)
            issue_in(k + 6, f2, sf2)
            wait_sc(f3, k + 3, ss3)
            issue_in(k + 7, f3, sf3)
            return carry

        lax.fori_loop(0, 18, quadbody, 0)
        wait_in(72, f0, sf0)
        issue_sc(f0, 72, ss0)
        wait_in(73, f1, sf1)
        issue_sc(f1, 73, ss1)
        wait_in(74, f2, sf2)
        issue_sc(f2, 74, ss2)
        wait_in(75, f3, sf3)
        issue_sc(f3, 75, ss3)
        wait_sc(f0, 72, ss0)
        issue_in(76, f0, sf0)
        wait_sc(f1, 73, ss1)
        issue_in(77, f1, sf1)
        wait_sc(f2, 74, ss2)
        wait_sc(f3, 75, ss3)
        wait_in(76, f0, sf0)
        issue_sc(f0, 76, ss0)
        wait_in(77, f1, sf1)
        issue_sc(f1, 77, ss1)
        wait_sc(f0, 76, ss0)
        wait_sc(f1, 77, ss1)

        @pl.when(sid < TAIL // CH)
        def _tail():
            base = pl.multiple_of(NS * TROWS + sid * CH, CH)
            pltpu.sync_copy(feat_hbm.at[pl.ds(base, CH), pl.ds(col0, COLS)],
                            f0)
            pltpu.sync_copy(lab_hbm.at[pl.ds(base, CH)],
                            lab_all.at[pl.ds(0, CH)])
            lax.fori_loop(0, CH // L, remap, 0)
            pltpu.sync_copy(f0, accum.at[ids_at(0)], add=True)

        plsc.subcore_barrier()

        @pl.when(sid < NFT)
        def _finalize():
            pltpu.sync_copy(accum.at[pl.ds(r0, RPT), :],
                            out_hbm.at[pl.ds(r0, RPT), pl.ds(col0, COLS)])

    return body(features, labels)


def _histo_kernel(lab_ref, out_ref):
    g = pl.program_id(0)
    lab = lab_ref[...]
    rows = []
    for i in range(BPG):
        b = g * BPG + i
        cnt = jnp.sum((lab == (b + 3)).astype(jnp.float32))
        rows.append(jnp.full((D,), cnt, jnp.float32))
    out_ref[...] = jnp.stack(rows, axis=0)


def _tc_histogram(labels2d):
    return pl.pallas_call(
        _histo_kernel,
        grid=(NB // BPG,),
        in_specs=[pl.BlockSpec(labels2d.shape, lambda g: (0, 0))],
        out_specs=pl.BlockSpec((BPG, D), lambda g: (g, 0)),
        out_shape=jax.ShapeDtypeStruct((NB, D), jnp.float32),
    )(labels2d)


def _div_kernel(sum_ref, cnt_ref, out_ref):
    s = sum_ref[...]
    c = cnt_ref[...]
    out_ref[...] = jnp.where(c > 0.0, s / jnp.where(c > 0.0, c, 1.0), 0.0)


def _tc_divide(sums, cnts):
    return pl.pallas_call(
        _div_kernel,
        in_specs=[pl.BlockSpec(sums.shape, lambda: (0, 0)),
                  pl.BlockSpec(cnts.shape, lambda: (0, 0))],
        out_specs=pl.BlockSpec(sums.shape, lambda: (0, 0)),
        out_shape=jax.ShapeDtypeStruct(sums.shape, jnp.float32),
    )(sums, cnts)


def kernel(features, labels):
    sums = _sc_segment_sum(features, labels)
    cnts = _tc_histogram(labels.reshape(N // CH, CH))
    center = _tc_divide(sums, cnts)
    return center[:DUMP]
